# baseline (speedup 1.0000x reference)
import functools

import jax
import jax.numpy as jnp
from jax import lax
from jax.experimental import pallas as pl
from jax.experimental.pallas import tpu as pltpu
from jax.experimental.pallas import tpu_sc as plsc

T = 256
FB = 1024


def _routing_metadata(expert_indices, expert_weights, E, T, G_MAX, T_MAX):
    P = expert_indices.size
    e = expert_indices.reshape(P).astype(jnp.int32)
    w = expert_weights.reshape(P)
    topk = expert_indices.shape[-1]
    tok = (jnp.arange(P, dtype=jnp.int32) // topk).astype(jnp.int32)

    onehot = (e[:, None] == jnp.arange(E, dtype=jnp.int32)[None, :]).astype(jnp.int32)
    counts = jnp.sum(onehot, axis=0)
    ranks = jnp.cumsum(onehot, axis=0) - onehot
    rank = jnp.take_along_axis(ranks, e[:, None], axis=1)[:, 0]

    tiles_per_e = (counts + T - 1) // T
    tile_start = jnp.concatenate(
        [jnp.zeros((1,), jnp.int32), jnp.cumsum(tiles_per_e).astype(jnp.int32)])
    padded_start = tile_start[:-1] * T
    pos = padded_start[e] + rank

    P_MAX = G_MAX * T
    row_token = jnp.zeros((P_MAX,), jnp.int32).at[pos].set(tok)
    row_weight = jnp.zeros((P_MAX,), jnp.float32).at[pos].set(w)

    t_ids = jnp.arange(T_MAX, dtype=jnp.int32)
    n_e = tiles_per_e.astype(jnp.int32)
    g_of_et = tile_start[:-1][:, None] + jnp.minimum(t_ids[None, :],
                                                     n_e[:, None] - 1)
    g_of_et = jnp.clip(g_of_et, 0, G_MAX - 1).reshape(-1)
    valid_et = (t_ids[None, :] < n_e[:, None]).astype(jnp.int32).reshape(-1)
    return row_token, row_weight, pos, g_of_et, valid_et


def _make_sc_gather(P_MAX, H):
    info = plsc.get_sparse_core_info()
    NW = info.num_cores * info.num_subcores
    b_per_w = P_MAX // NW
    CHUNK = 24
    chunks = []
    off = 0
    while off < b_per_w:
        c = min(CHUNK, b_per_w - off)
        chunks.append((off, c))
        off += c
    mesh = plsc.VectorSubcoreMesh(core_axis_name="c", subcore_axis_name="s")

    NBUF = 4
    nch = len(chunks)

    @functools.partial(
        pl.kernel,
        out_type=jax.ShapeDtypeStruct((P_MAX, H), jnp.float32),
        mesh=mesh,
        scratch_types=[
            pltpu.VMEM((b_per_w,), jnp.int32),
            pltpu.VMEM((NBUF, CHUNK, H), jnp.float32),
            [pltpu.SemaphoreType.DMA] * NBUF,
            [pltpu.SemaphoreType.DMA] * NBUF,
        ],
    )
    def gather_k(x_hbm, idx_hbm, out_hbm, idx_v, rows_v, gsems, ssems):
        wid = lax.axis_index("s") * info.num_cores + lax.axis_index("c")
        base = wid * b_per_w

        def start_gather(i):
            off, c = chunks[i]
            return pltpu.async_copy(
                x_hbm.at[idx_v.at[pl.ds(off, c)]],
                rows_v.at[i % NBUF, pl.ds(0, c)], gsems[i % NBUF])

        def start_store(i):
            off, c = chunks[i]
            return pltpu.async_copy(
                rows_v.at[i % NBUF, pl.ds(0, c)],
                out_hbm.at[pl.ds(base + off, c)], ssems[i % NBUF])

        pltpu.sync_copy(idx_hbm.at[pl.ds(base, b_per_w)], idx_v)
        gops = {}
        sops = {}
        for i in range(min(NBUF - 1, nch)):
            gops[i] = start_gather(i)
        for i in range(nch):
            gops[i].wait()
            sops[i] = start_store(i)
            k = i + NBUF - 1
            if k < nch:
                if k - NBUF >= 0:
                    sops[k - NBUF].wait()
                gops[k] = start_gather(k)
        for i in range(max(0, nch - NBUF), nch):
            if i in sops:
                sops[i].wait()

    return gather_k


def _make_sc_combine(S, H, P_MAX):
    info = plsc.get_sparse_core_info()
    NW = info.num_cores * info.num_subcores
    t_per_w = S // NW
    CH = 32
    n_ch = t_per_w // CH
    L = info.num_lanes
    mesh = plsc.VectorSubcoreMesh(core_axis_name="c", subcore_axis_name="s")

    @functools.partial(
        pl.kernel,
        out_type=jax.ShapeDtypeStruct((S, H), jnp.float32),
        mesh=mesh,
        scratch_types=[
            pltpu.VMEM((CH,), jnp.int32),
            pltpu.VMEM((CH,), jnp.int32),
            pltpu.VMEM((CH, H), jnp.float32),
            pltpu.VMEM((CH, H), jnp.float32),
            pltpu.SemaphoreType.DMA,
            pltpu.SemaphoreType.DMA,
        ],
    )
    def combine_k(yg_hbm, pos0_hbm, pos1_hbm, out_hbm,
                  i0_v, i1_v, a_v, b_v, sem0, sem1):
        wid = lax.axis_index("s") * info.num_cores + lax.axis_index("c")
        base = wid * t_per_w
        for c in range(n_ch):
            cbase = base + c * CH
            pltpu.sync_copy(pos0_hbm.at[pl.ds(cbase, CH)], i0_v)
            pltpu.sync_copy(pos1_hbm.at[pl.ds(cbase, CH)], i1_v)
            cp0 = pltpu.async_copy(yg_hbm.at[i0_v], a_v, sem0)
            cp1 = pltpu.async_copy(yg_hbm.at[i1_v], b_v, sem1)
            cp0.wait()
            cp1.wait()

            def add_row(r, _):
                def add_vec(v, _):
                    sl = pl.ds(v * L, L)
                    a_v[r, sl] = a_v[r, sl] + b_v[r, sl]
                    return 0
                return lax.fori_loop(0, H // L, add_vec, 0)

            lax.fori_loop(0, CH, add_row, 0)
            pltpu.sync_copy(a_v, out_hbm.at[pl.ds(cbase, CH)])

    return combine_k


def _ffn_body(ge_ref, ve_ref, xg_ref, gw_ref, uw_ref, dw_ref, w_ref, out_ref,
              acc_ref, *, n_ff_blocks, t_max):
    e = pl.program_id(0)
    j = pl.program_id(1)
    t = pl.program_id(2)

    @pl.when(ve_ref[e * t_max + t] == 1)
    def _():
        x = xg_ref[...]
        gate = lax.dot_general(
            x, gw_ref[0], (((1,), (1,)), ((), ())),
            preferred_element_type=jnp.float32)
        up = lax.dot_general(
            x, uw_ref[0], (((1,), (1,)), ((), ())),
            preferred_element_type=jnp.float32)
        inter = (gate * jax.nn.sigmoid(gate)) * up
        part = lax.dot_general(
            inter, dw_ref[0], (((1,), (1,)), ((), ())),
            preferred_element_type=jnp.float32)

        @pl.when(j == 0)
        def _():
            acc_ref[t] = part

        @pl.when((j > 0) & (j < n_ff_blocks - 1))
        def _():
            acc_ref[t] = acc_ref[t] + part

        @pl.when(j == n_ff_blocks - 1)
        def _():
            out_ref[...] = (acc_ref[t] + part) * w_ref[...]


def kernel(x, expert_indices, expert_weights, gate_proj, up_proj, down_proj):
    b, s, h = x.shape
    E, FF, _ = gate_proj.shape
    topk = expert_indices.shape[-1]
    P = b * s * topk
    G_MAX = P // T + (E - 1)
    P_MAX = G_MAX * T
    J = FF // FB

    T_MAX = -(-P // T)
    x_flat = x.reshape(b * s, h)
    row_token, row_weight, pos, g_of_et, valid_et = _routing_metadata(
        expert_indices, expert_weights, E, T, G_MAX, T_MAX)

    xg = _make_sc_gather(P_MAX, h)(x_flat, row_token)

    def out_idx(e, j, t, ge, ve):
        ok = (j == J - 1) & (ve[e * T_MAX + t] == 1)
        return (jnp.where(ok, ge[e * T_MAX + t], G_MAX), 0)

    grid_spec = pltpu.PrefetchScalarGridSpec(
        num_scalar_prefetch=2,
        grid=(E, J, T_MAX),
        in_specs=[
            pl.BlockSpec((T, h), lambda e, j, t, ge, ve: (ge[e * T_MAX + t], 0)),
            pl.BlockSpec((1, FB, h), lambda e, j, t, ge, ve: (e, j, 0)),
            pl.BlockSpec((1, FB, h), lambda e, j, t, ge, ve: (e, j, 0)),
            pl.BlockSpec((1, h, FB), lambda e, j, t, ge, ve: (e, 0, j)),
            pl.BlockSpec((T, 1), lambda e, j, t, ge, ve: (ge[e * T_MAX + t], 0)),
        ],
        out_specs=pl.BlockSpec((T, h), out_idx),
        scratch_shapes=[pltpu.VMEM((T_MAX, T, h), jnp.float32)],
    )
    yg = pl.pallas_call(
        functools.partial(_ffn_body, n_ff_blocks=J, t_max=T_MAX),
        grid_spec=grid_spec,
        out_shape=jax.ShapeDtypeStruct(((G_MAX + 1) * T, h), jnp.float32),
        compiler_params=pltpu.CompilerParams(
            dimension_semantics=("arbitrary", "arbitrary", "arbitrary")),
    )(g_of_et, valid_et, xg, gate_proj, up_proj, down_proj,
      row_weight.reshape(P_MAX, 1))

    pos2 = pos.reshape(b * s, topk)
    out = _make_sc_combine(b * s, h, (G_MAX + 1) * T)(
        yg, pos2[:, 0].astype(jnp.int32), pos2[:, 1].astype(jnp.int32))
    return out.reshape(b, s, h)


# speedup vs baseline: 1.3647x; 1.3647x over previous
import functools

import jax
import jax.numpy as jnp
from jax import lax
from jax.experimental import pallas as pl
from jax.experimental.pallas import tpu as pltpu
from jax.experimental.pallas import tpu_sc as plsc

T = 256
FB = 1024


def _routing_metadata(expert_indices, expert_weights, E, T, G_MAX, T_MAX, n_ff):
    P = expert_indices.size
    e = expert_indices.reshape(P).astype(jnp.int32)
    w = expert_weights.reshape(P)
    topk = expert_indices.shape[-1]
    tok = (jnp.arange(P, dtype=jnp.int32) // topk).astype(jnp.int32)

    onehot = (e[:, None] == jnp.arange(E, dtype=jnp.int32)[None, :]).astype(jnp.int32)
    counts = jnp.sum(onehot, axis=0)
    ranks = jnp.cumsum(onehot, axis=0) - onehot
    rank = jnp.take_along_axis(ranks, e[:, None], axis=1)[:, 0]

    tiles_per_e = (counts + T - 1) // T
    tile_start = jnp.concatenate(
        [jnp.zeros((1,), jnp.int32), jnp.cumsum(tiles_per_e).astype(jnp.int32)])
    padded_start = tile_start[:-1] * T
    pos = padded_start[e] + rank

    P_MAX = G_MAX * T
    row_token = jnp.zeros((P_MAX,), jnp.int32).at[pos].set(tok)
    row_weight = jnp.zeros((P_MAX,), jnp.float32).at[pos].set(w)

    J = n_ff
    n_e = tiles_per_e.astype(jnp.int32)
    N_STEPS = G_MAX * J
    e_ids = jnp.arange(E, dtype=jnp.int32)[:, None, None]
    j_ids = jnp.arange(J, dtype=jnp.int32)[None, :, None]
    t_ids = jnp.arange(T_MAX, dtype=jnp.int32)[None, None, :]
    valid = (t_ids < n_e[:, None, None])
    dest = (J * tile_start[:-1][:, None, None] + j_ids * n_e[:, None, None]
            + t_ids)
    dest = jnp.where(valid, dest, N_STEPS).reshape(-1)
    src_e = jnp.broadcast_to(e_ids, (E, J, T_MAX)).reshape(-1)
    src_j = jnp.broadcast_to(j_ids, (E, J, T_MAX)).reshape(-1)
    src_t = jnp.broadcast_to(t_ids, (E, J, T_MAX)).reshape(-1)
    src_g = jnp.broadcast_to(tile_start[:-1][:, None, None] + t_ids,
                             (E, J, T_MAX)).reshape(-1)
    zeros = jnp.zeros((N_STEPS,), jnp.int32)
    sched_e = zeros.at[dest].set(src_e, mode="drop")
    sched_j = zeros.at[dest].set(src_j, mode="drop")
    sched_t = zeros.at[dest].set(src_t, mode="drop")
    sched_g = zeros.at[dest].set(src_g, mode="drop")
    n_steps = tile_start[E] * J
    k_ids = jnp.arange(N_STEPS, dtype=jnp.int32)
    live = k_ids < n_steps
    last = jnp.maximum(n_steps - 1, 0)
    sched_e = jnp.where(live, sched_e, sched_e[last])
    sched_j = jnp.where(live, sched_j, sched_j[last])
    sched_t = jnp.where(live, sched_t, sched_t[last])
    sched_g = jnp.where(live, sched_g, sched_g[last])
    sched_valid = live.astype(jnp.int32)
    sched = jnp.stack([sched_e, sched_j, sched_t, sched_g, sched_valid])
    return row_token, row_weight, pos, sched


def _make_sc_gather(P_MAX, H):
    info = plsc.get_sparse_core_info()
    NW = info.num_cores * info.num_subcores
    b_per_w = P_MAX // NW
    CHUNK = 56
    chunks = []
    off = 0
    while off < b_per_w:
        c = min(CHUNK, b_per_w - off)
        chunks.append((off, c))
        off += c
    mesh = plsc.VectorSubcoreMesh(core_axis_name="c", subcore_axis_name="s")

    NBUF = 2
    nch = len(chunks)

    @functools.partial(
        pl.kernel,
        out_type=jax.ShapeDtypeStruct((P_MAX, H), jnp.float32),
        mesh=mesh,
        scratch_types=[
            pltpu.VMEM((b_per_w,), jnp.int32),
            pltpu.VMEM((NBUF, CHUNK, H), jnp.float32),
            [pltpu.SemaphoreType.DMA] * NBUF,
            [pltpu.SemaphoreType.DMA] * NBUF,
        ],
    )
    def gather_k(x_hbm, idx_hbm, out_hbm, idx_v, rows_v, gsems, ssems):
        wid = lax.axis_index("s") * info.num_cores + lax.axis_index("c")
        base = wid * b_per_w

        def start_gather(i):
            off, c = chunks[i]
            return pltpu.async_copy(
                x_hbm.at[idx_v.at[pl.ds(off, c)]],
                rows_v.at[i % NBUF, pl.ds(0, c)], gsems[i % NBUF])

        def start_store(i):
            off, c = chunks[i]
            return pltpu.async_copy(
                rows_v.at[i % NBUF, pl.ds(0, c)],
                out_hbm.at[pl.ds(base + off, c)], ssems[i % NBUF])

        pltpu.sync_copy(idx_hbm.at[pl.ds(base, b_per_w)], idx_v)
        gops = {}
        sops = {}
        for i in range(min(NBUF - 1, nch)):
            gops[i] = start_gather(i)
        for i in range(nch):
            gops[i].wait()
            sops[i] = start_store(i)
            k = i + NBUF - 1
            if k < nch:
                if k - NBUF >= 0:
                    sops[k - NBUF].wait()
                gops[k] = start_gather(k)
        for i in range(max(0, nch - NBUF), nch):
            if i in sops:
                sops[i].wait()

    return gather_k


def _make_sc_combine(S, H, P_MAX):
    info = plsc.get_sparse_core_info()
    NW = info.num_cores * info.num_subcores
    t_per_w = S // NW
    CH = 32
    n_ch = t_per_w // CH
    L = info.num_lanes
    mesh = plsc.VectorSubcoreMesh(core_axis_name="c", subcore_axis_name="s")

    @functools.partial(
        pl.kernel,
        out_type=jax.ShapeDtypeStruct((S, H), jnp.float32),
        mesh=mesh,
        scratch_types=[
            pltpu.VMEM((CH,), jnp.int32),
            pltpu.VMEM((CH,), jnp.int32),
            pltpu.VMEM((CH, H), jnp.float32),
            pltpu.VMEM((CH, H), jnp.float32),
            pltpu.SemaphoreType.DMA,
            pltpu.SemaphoreType.DMA,
        ],
    )
    def combine_k(yg_hbm, pos0_hbm, pos1_hbm, out_hbm,
                  i0_v, i1_v, a_v, b_v, sem0, sem1):
        wid = lax.axis_index("s") * info.num_cores + lax.axis_index("c")
        base = wid * t_per_w
        for c in range(n_ch):
            cbase = base + c * CH
            pltpu.sync_copy(pos0_hbm.at[pl.ds(cbase, CH)], i0_v)
            pltpu.sync_copy(pos1_hbm.at[pl.ds(cbase, CH)], i1_v)
            cp0 = pltpu.async_copy(yg_hbm.at[i0_v], a_v, sem0)
            cp1 = pltpu.async_copy(yg_hbm.at[i1_v], b_v, sem1)
            cp0.wait()
            cp1.wait()

            def add_row(r, _):
                def add_vec(v, _):
                    sl = pl.ds(v * L, L)
                    a_v[r, sl] = a_v[r, sl] + b_v[r, sl]
                    return 0
                return lax.fori_loop(0, H // L, add_vec, 0)

            lax.fori_loop(0, CH, add_row, 0)
            pltpu.sync_copy(a_v, out_hbm.at[pl.ds(cbase, CH)])

    return combine_k


def _ffn_body(sched_ref, xg_ref, gw_ref, uw_ref, dw_ref, w_ref, out_ref,
              acc_ref, *, n_ff_blocks):
    k = pl.program_id(0)
    j = sched_ref[1, k]
    t = sched_ref[2, k]

    @pl.when(sched_ref[4, k] == 1)
    def _():
        x = xg_ref[...]
        gate = lax.dot_general(
            x, gw_ref[0], (((1,), (1,)), ((), ())),
            preferred_element_type=jnp.float32)
        up = lax.dot_general(
            x, uw_ref[0], (((1,), (1,)), ((), ())),
            preferred_element_type=jnp.float32)
        inter = (gate * jax.nn.sigmoid(gate)) * up
        part = lax.dot_general(
            inter, dw_ref[0], (((1,), (1,)), ((), ())),
            preferred_element_type=jnp.float32)

        @pl.when(j == 0)
        def _():
            acc_ref[t] = part

        @pl.when((j > 0) & (j < n_ff_blocks - 1))
        def _():
            acc_ref[t] = acc_ref[t] + part

        @pl.when(j == n_ff_blocks - 1)
        def _():
            out_ref[...] = (acc_ref[t] + part) * w_ref[...]


def kernel(x, expert_indices, expert_weights, gate_proj, up_proj, down_proj):
    b, s, h = x.shape
    E, FF, _ = gate_proj.shape
    topk = expert_indices.shape[-1]
    P = b * s * topk
    G_MAX = P // T + (E - 1)
    P_MAX = G_MAX * T
    J = FF // FB

    T_MAX = -(-P // T)
    N_STEPS = G_MAX * J
    x_flat = x.reshape(b * s, h)
    row_token, row_weight, pos, sched = _routing_metadata(
        expert_indices, expert_weights, E, T, G_MAX, T_MAX, J)

    xg = _make_sc_gather(P_MAX, h)(x_flat, row_token)

    def out_idx(k, sc):
        ok = (sc[1, k] == J - 1) & (sc[4, k] == 1)
        return (jnp.where(ok, sc[3, k], G_MAX), 0)

    grid_spec = pltpu.PrefetchScalarGridSpec(
        num_scalar_prefetch=1,
        grid=(N_STEPS,),
        in_specs=[
            pl.BlockSpec((T, h), lambda k, sc: (sc[3, k], 0)),
            pl.BlockSpec((1, FB, h), lambda k, sc: (sc[0, k], sc[1, k], 0)),
            pl.BlockSpec((1, FB, h), lambda k, sc: (sc[0, k], sc[1, k], 0)),
            pl.BlockSpec((1, h, FB), lambda k, sc: (sc[0, k], 0, sc[1, k])),
            pl.BlockSpec((T, 1), lambda k, sc: (sc[3, k], 0)),
        ],
        out_specs=pl.BlockSpec((T, h), out_idx),
        scratch_shapes=[pltpu.VMEM((T_MAX, T, h), jnp.float32)],
    )
    yg = pl.pallas_call(
        functools.partial(_ffn_body, n_ff_blocks=J),
        grid_spec=grid_spec,
        out_shape=jax.ShapeDtypeStruct(((G_MAX + 1) * T, h), jnp.float32),
        compiler_params=pltpu.CompilerParams(
            dimension_semantics=("arbitrary",)),
    )(sched, xg, gate_proj, up_proj, down_proj,
      row_weight.reshape(P_MAX, 1))

    pos2 = pos.reshape(b * s, topk)
    out = _make_sc_combine(b * s, h, (G_MAX + 1) * T)(
        yg, pos2[:, 0].astype(jnp.int32), pos2[:, 1].astype(jnp.int32))
    return out.reshape(b, s, h)


# speedup vs baseline: 1.9301x; 1.4143x over previous
import functools

import jax
import jax.numpy as jnp
from jax import lax
from jax.experimental import pallas as pl
from jax.experimental.pallas import tpu as pltpu
from jax.experimental.pallas import tpu_sc as plsc

T = 256
FB = 1024


def _routing_metadata(expert_indices, expert_weights, E, T, G_MAX, T_MAX, n_ff):
    P = expert_indices.size
    e = expert_indices.reshape(P).astype(jnp.int32)
    w = expert_weights.reshape(P)
    topk = expert_indices.shape[-1]
    tok = (jnp.arange(P, dtype=jnp.int32) // topk).astype(jnp.int32)

    onehot = (e[:, None] == jnp.arange(E, dtype=jnp.int32)[None, :]).astype(jnp.int32)
    counts = jnp.sum(onehot, axis=0)
    ranks = jnp.cumsum(onehot, axis=0) - onehot
    rank = jnp.take_along_axis(ranks, e[:, None], axis=1)[:, 0]

    tiles_per_e = (counts + T - 1) // T
    tile_start = jnp.concatenate(
        [jnp.zeros((1,), jnp.int32), jnp.cumsum(tiles_per_e).astype(jnp.int32)])
    padded_start = tile_start[:-1] * T
    pos = padded_start[e] + rank

    del w

    J = n_ff
    n_e = tiles_per_e.astype(jnp.int32)
    N_STEPS = G_MAX * J
    e_ids = jnp.arange(E, dtype=jnp.int32)[:, None, None]
    j_ids = jnp.arange(J, dtype=jnp.int32)[None, :, None]
    t_ids = jnp.arange(T_MAX, dtype=jnp.int32)[None, None, :]
    valid = (t_ids < n_e[:, None, None])
    dest = (J * tile_start[:-1][:, None, None] + j_ids * n_e[:, None, None]
            + t_ids)
    dest = jnp.where(valid, dest, N_STEPS).reshape(-1)
    src_e = jnp.broadcast_to(e_ids, (E, J, T_MAX)).reshape(-1)
    src_j = jnp.broadcast_to(j_ids, (E, J, T_MAX)).reshape(-1)
    src_t = jnp.broadcast_to(t_ids, (E, J, T_MAX)).reshape(-1)
    src_g = jnp.broadcast_to(tile_start[:-1][:, None, None] + t_ids,
                             (E, J, T_MAX)).reshape(-1)
    zeros = jnp.zeros((N_STEPS,), jnp.int32)
    sched_e = zeros.at[dest].set(src_e, mode="drop")
    sched_j = zeros.at[dest].set(src_j, mode="drop")
    sched_t = zeros.at[dest].set(src_t, mode="drop")
    sched_g = zeros.at[dest].set(src_g, mode="drop")
    n_steps = tile_start[E] * J
    k_ids = jnp.arange(N_STEPS, dtype=jnp.int32)
    live = k_ids < n_steps
    last = jnp.maximum(n_steps - 1, 0)
    sched_e = jnp.where(live, sched_e, sched_e[last])
    sched_j = jnp.where(live, sched_j, sched_j[last])
    sched_t = jnp.where(live, sched_t, sched_t[last])
    sched_g = jnp.where(live, sched_g, sched_g[last])
    sched_valid = live.astype(jnp.int32)
    grp_first = jnp.concatenate(
        [jnp.ones((1,), jnp.int32),
         ((sched_e[1:] != sched_e[:-1]) | (sched_j[1:] != sched_j[:-1])
          ).astype(jnp.int32) * sched_valid[1:]])
    gid = jnp.cumsum(grp_first) - 1
    nz = (n_e > 0).astype(jnp.int32)
    n_groups = jnp.sum(nz) * J
    nonempty = jnp.zeros((E,), jnp.int32).at[
        jnp.where(nz == 1, jnp.cumsum(nz) - 1, E)].set(
            jnp.arange(E, dtype=jnp.int32), mode="drop")
    nxt = jnp.minimum(gid + 1, jnp.maximum(n_groups - 1, 0))
    pf_e = nonempty[nxt // J]
    pf_j = (nxt % J).astype(jnp.int32)
    pf_valid = ((gid + 1) < n_groups).astype(jnp.int32) * grp_first
    cur_buf = (gid % 2).astype(jnp.int32)
    sched = jnp.stack([sched_e, sched_j, sched_t, sched_g, sched_valid,
                       grp_first, cur_buf, pf_valid, pf_e, pf_j])
    return tok, pos, sched


def _make_sc_dispatch(P, P_MAX, H):
    info = plsc.get_sparse_core_info()
    NW = info.num_cores * info.num_subcores
    b_per_w = P // NW
    L = info.num_lanes
    CHUNK = 32
    nch = b_per_w // CHUNK
    NBUF = 3
    mesh = plsc.VectorSubcoreMesh(core_axis_name="c", subcore_axis_name="s")

    @functools.partial(
        pl.kernel,
        out_type=jax.ShapeDtypeStruct((P_MAX, H), jnp.float32),
        mesh=mesh,
        scratch_types=[
            [pltpu.VMEM((CHUNK,), jnp.int32) for _ in range(nch)],
            [pltpu.VMEM((CHUNK,), jnp.int32) for _ in range(nch)],
            pltpu.VMEM((NBUF, CHUNK, H), jnp.float32),
            [pltpu.SemaphoreType.DMA] * NBUF,
            [pltpu.SemaphoreType.DMA] * NBUF,
        ],
    )
    def dispatch_k(x_hbm, tok_hbm, pos_hbm, xg_hbm,
                   tok_vs, pos_vs, rows_v, gsems, ssems):
        wid = lax.axis_index("s") * info.num_cores + lax.axis_index("c")
        base = wid * b_per_w

        def start_gather(i):
            return pltpu.async_copy(
                x_hbm.at[tok_vs[i]], rows_v.at[i % NBUF], gsems[i % NBUF])

        def start_scatter(i):
            return pltpu.async_copy(
                rows_v.at[i % NBUF], xg_hbm.at[pos_vs[i]], ssems[i % NBUF])

        for i in range(nch):
            off = base + i * CHUNK
            pltpu.sync_copy(tok_hbm.at[pl.ds(off, CHUNK)], tok_vs[i])
            pltpu.sync_copy(pos_hbm.at[pl.ds(off, CHUNK)], pos_vs[i])
        gops = {}
        sops = {}
        for i in range(min(NBUF - 1, nch)):
            gops[i] = start_gather(i)
        for i in range(nch):
            gops[i].wait()
            sops[i] = start_scatter(i)
            k = i + NBUF - 1
            if k < nch:
                if k - NBUF >= 0:
                    sops[k - NBUF].wait()
                gops[k] = start_gather(k)
        for i in range(max(0, nch - NBUF), nch):
            if i in sops:
                sops[i].wait()

    return dispatch_k


def _make_sc_combine(S, H, P_MAX):
    info = plsc.get_sparse_core_info()
    NW = info.num_cores * info.num_subcores
    t_per_w = S // NW
    CH = 32
    n_ch = t_per_w // CH
    L = info.num_lanes
    mesh = plsc.VectorSubcoreMesh(core_axis_name="c", subcore_axis_name="s")

    @functools.partial(
        pl.kernel,
        out_type=jax.ShapeDtypeStruct((S, H), jnp.float32),
        mesh=mesh,
        scratch_types=[
            pltpu.VMEM((CH,), jnp.int32),
            pltpu.VMEM((CH,), jnp.int32),
            pltpu.VMEM((CH, H), jnp.float32),
            pltpu.VMEM((CH, H), jnp.float32),
            pltpu.SemaphoreType.DMA,
            pltpu.SemaphoreType.DMA,
        ],
    )
    def combine_k(yg_hbm, pos0_hbm, pos1_hbm, out_hbm,
                  i0_v, i1_v, a_v, b_v, sem0, sem1):
        wid = lax.axis_index("s") * info.num_cores + lax.axis_index("c")
        base = wid * t_per_w
        for c in range(n_ch):
            cbase = base + c * CH
            pltpu.sync_copy(pos0_hbm.at[pl.ds(cbase, CH)], i0_v)
            pltpu.sync_copy(pos1_hbm.at[pl.ds(cbase, CH)], i1_v)
            cp0 = pltpu.async_copy(yg_hbm.at[i0_v], a_v, sem0)
            cp1 = pltpu.async_copy(yg_hbm.at[i1_v], b_v, sem1)
            cp0.wait()
            cp1.wait()

            def add_row(r, _):
                def add_vec(v, _):
                    sl = pl.ds(v * L, L)
                    a_v[r, sl] = a_v[r, sl] + b_v[r, sl]
                    return 0
                return lax.fori_loop(0, H // L, add_vec, 0)

            lax.fori_loop(0, CH, add_row, 0)
            pltpu.sync_copy(a_v, out_hbm.at[pl.ds(cbase, CH)])

    return combine_k


def _ffn_body(sched_ref, xg_ref, gw_hbm, uw_hbm, dw_hbm, w_ref, out_ref,
              acc_ref, gsc, usc, dsc, gsem, usem, dsem,
              *, n_ff_blocks, fb):
    k = pl.program_id(0)
    j = sched_ref[1, k]
    t = sched_ref[2, k]
    buf = sched_ref[6, k]

    def wcopies(slot, pe, pj):
        return (
            pltpu.make_async_copy(
                gw_hbm.at[pe, pl.ds(pj * fb, fb), :], gsc.at[slot],
                gsem.at[slot]),
            pltpu.make_async_copy(
                uw_hbm.at[pe, pl.ds(pj * fb, fb), :], usc.at[slot],
                usem.at[slot]),
            pltpu.make_async_copy(
                dw_hbm.at[pe, :, pl.ds(pj * fb, fb)], dsc.at[slot],
                dsem.at[slot]),
        )

    @pl.when(sched_ref[5, k] == 1)
    def _():
        @pl.when(k == 0)
        def _():
            for cp in wcopies(buf, sched_ref[0, k], sched_ref[1, k]):
                cp.start()

        for cp in wcopies(buf, sched_ref[0, k], sched_ref[1, k]):
            cp.wait()

        @pl.when(sched_ref[7, k] == 1)
        def _():
            for cp in wcopies(1 - buf, sched_ref[8, k], sched_ref[9, k]):
                cp.start()

    @pl.when(sched_ref[4, k] == 1)
    def _():
        x = xg_ref[...]
        gate = lax.dot_general(
            x, gsc.at[buf][...], (((1,), (1,)), ((), ())),
            preferred_element_type=jnp.float32)
        up = lax.dot_general(
            x, usc.at[buf][...], (((1,), (1,)), ((), ())),
            preferred_element_type=jnp.float32)
        inter = (gate * jax.nn.sigmoid(gate)) * up
        part = lax.dot_general(
            inter, dsc.at[buf][...], (((1,), (1,)), ((), ())),
            preferred_element_type=jnp.float32)

        @pl.when(j == 0)
        def _():
            acc_ref[t] = part

        @pl.when((j > 0) & (j < n_ff_blocks - 1))
        def _():
            acc_ref[t] = acc_ref[t] + part

        @pl.when(j == n_ff_blocks - 1)
        def _():
            out_ref[...] = (acc_ref[t] + part) * w_ref[...]


def kernel(x, expert_indices, expert_weights, gate_proj, up_proj, down_proj):
    b, s, h = x.shape
    E, FF, _ = gate_proj.shape
    topk = expert_indices.shape[-1]
    P = b * s * topk
    G_MAX = P // T + (E - 1)
    P_MAX = G_MAX * T
    J = FF // FB

    T_MAX = -(-P // T)
    N_STEPS = G_MAX * J
    x_flat = x.reshape(b * s, h)
    tok, pos, sched = _routing_metadata(
        expert_indices, expert_weights, E, T, G_MAX, T_MAX, J)

    xg = _make_sc_dispatch(P, P_MAX, h)(x_flat, tok, pos.astype(jnp.int32))
    row_w = jnp.zeros((P_MAX,), jnp.float32).at[pos].set(
        expert_weights.reshape(P))

    def out_idx(k, sc):
        ok = (sc[1, k] == J - 1) & (sc[4, k] == 1)
        return (jnp.where(ok, sc[3, k], G_MAX), 0)

    grid_spec = pltpu.PrefetchScalarGridSpec(
        num_scalar_prefetch=1,
        grid=(N_STEPS,),
        in_specs=[
            pl.BlockSpec((T, h), lambda k, sc: (sc[3, k], 0)),
            pl.BlockSpec(memory_space=pl.ANY),
            pl.BlockSpec(memory_space=pl.ANY),
            pl.BlockSpec(memory_space=pl.ANY),
            pl.BlockSpec((T, 1), lambda k, sc: (sc[3, k], 0)),
        ],
        out_specs=pl.BlockSpec((T, h), out_idx),
        scratch_shapes=[
            pltpu.VMEM((T_MAX, T, h), jnp.float32),
            pltpu.VMEM((2, FB, h), jnp.float32),
            pltpu.VMEM((2, FB, h), jnp.float32),
            pltpu.VMEM((2, h, FB), jnp.float32),
            pltpu.SemaphoreType.DMA((2,)),
            pltpu.SemaphoreType.DMA((2,)),
            pltpu.SemaphoreType.DMA((2,)),
        ],
    )
    yg = pl.pallas_call(
        functools.partial(_ffn_body, n_ff_blocks=J, fb=FB),
        grid_spec=grid_spec,
        out_shape=jax.ShapeDtypeStruct(((G_MAX + 1) * T, h), jnp.float32),
        compiler_params=pltpu.CompilerParams(
            dimension_semantics=("arbitrary",)),
    )(sched, xg, gate_proj, up_proj, down_proj, row_w.reshape(P_MAX, 1))

    pos2 = pos.reshape(b * s, topk)
    out = _make_sc_combine(b * s, h, (G_MAX + 1) * T)(
        yg, pos2[:, 0].astype(jnp.int32), pos2[:, 1].astype(jnp.int32))
    return out.reshape(b, s, h)
